# baseline (speedup 1.0000x reference)
import functools

import jax
import jax.numpy as jnp
from jax import lax
from jax.experimental import pallas as pl
from jax.experimental.pallas import tpu as pltpu
from jax.experimental.pallas import tpu_sc as plsc

B, C, O, L = 1024, 50, 10, 5
NMETA, D, H = 10, 128, 256

NC, NS = 2, 16
NW = NC * NS
CHUNK = 80
NCH = (C * B) // (NW * CHUNK)
CEN_PER_W = B // NW


def _pipe_gather(idx, table, out, base, nch, rows, buf, gsem, wsem):
    gets, puts = [], []
    gets.append(pltpu.async_copy(
        table.at[idx.at[0]], buf.at[0, pl.ds(0, rows)], gsem))
    for k in range(nch):
        gets[k].wait()
        puts.append(pltpu.async_copy(
            buf.at[k % 2, pl.ds(0, rows)],
            out.at[pl.ds(base + k * rows, rows)], wsem))
        if k + 1 < nch:
            if k >= 1:
                puts[k - 1].wait()
            gets.append(pltpu.async_copy(
                table.at[idx.at[k + 1]],
                buf.at[(k + 1) % 2, pl.ds(0, rows)], gsem))
    if nch >= 2:
        puts[nch - 2].wait()
    puts[nch - 1].wait()


def _ctx_body(temb_ref, cidx_ref, sidx_ref, ctx_out, cen_out,
              idx_v, sidx_v, buf, gsem, wsem):
    wid = lax.axis_index("s") * NC + lax.axis_index("c")
    pltpu.sync_copy(cidx_ref.at[wid], idx_v)
    pltpu.sync_copy(sidx_ref.at[wid], sidx_v)
    _pipe_gather(idx_v, temb_ref, ctx_out, wid * CHUNK * NCH, NCH, CHUNK,
                 buf, gsem, wsem)
    _pipe_gather(sidx_v, temb_ref, cen_out, wid * CEN_PER_W, 1, CEN_PER_W,
                 buf, gsem, wsem)


def _dec_gather_body(demb_ref, didx_ref, dec_out, idx_v, buf, gsem, wsem):
    wid = lax.axis_index("s") * NC + lax.axis_index("c")
    pltpu.sync_copy(didx_ref.at[wid], idx_v)
    _pipe_gather(idx_v, demb_ref, dec_out, wid * CHUNK * NCH, NCH, CHUNK,
                 buf, gsem, wsem)


def _sc_mesh():
    return plsc.VectorSubcoreMesh(core_axis_name="c", subcore_axis_name="s",
                                  num_cores=NC, num_subcores=NS)


def _sc_gather(token_emb, dec_token_emb, context_ids, lf_ids, sf_ids):
    cidx = jnp.transpose(context_ids).reshape(NW, NCH, CHUNK).astype(jnp.int32)
    didx = jnp.transpose(lf_ids, (2, 1, 0)).reshape(NW, NCH, CHUNK).astype(jnp.int32)
    sidx = sf_ids.reshape(NW, 1, CEN_PER_W).astype(jnp.int32)

    ctx_out, cen_out = pl.kernel(
        _ctx_body,
        out_type=[
            jax.ShapeDtypeStruct((C * B, D), jnp.float32),
            jax.ShapeDtypeStruct((B, D), jnp.float32),
        ],
        mesh=_sc_mesh(),
        scratch_types=[
            pltpu.VMEM((NCH, CHUNK), jnp.int32),
            pltpu.VMEM((1, CEN_PER_W), jnp.int32),
            pltpu.VMEM((2, CHUNK, D), jnp.float32),
            pltpu.SemaphoreType.DMA,
            pltpu.SemaphoreType.DMA,
        ],
    )(token_emb, cidx, sidx)

    dec_out = pl.kernel(
        _dec_gather_body,
        out_type=jax.ShapeDtypeStruct((L * O * B, D), jnp.float32),
        mesh=_sc_mesh(),
        scratch_types=[
            pltpu.VMEM((NCH, CHUNK), jnp.int32),
            pltpu.VMEM((2, CHUNK, D), jnp.float32),
            pltpu.SemaphoreType.DMA,
            pltpu.SemaphoreType.DMA,
        ],
    )(dec_token_emb, didx)
    return (ctx_out.reshape(C, B, D), dec_out.reshape(L, O, B, D), cen_out)


def _softplus(x):
    return jnp.maximum(x, 0.0) + jnp.log(1.0 + jnp.exp(-jnp.abs(x)))


def _enc_body(ctx_ref, center_ref, onehot_ref, fW_ref, memb_ref, fb_ref,
              uW_ref, ub_ref, vW_ref, vb_ref, mu_ref, sig_ref, a_ref, acc_ref):
    c = pl.program_id(0)

    @pl.when(c == 0)
    def _():
        m = jnp.dot(onehot_ref[...], memb_ref[...],
                    preferred_element_type=jnp.float32)
        a_ref[...] = (jnp.dot(center_ref[...], fW_ref[0:D, :],
                              preferred_element_type=jnp.float32)
                      + jnp.dot(m, fW_ref[D:2 * D, :],
                                preferred_element_type=jnp.float32)
                      + fb_ref[...])
        acc_ref[...] = jnp.zeros_like(acc_ref)

    x = jnp.dot(ctx_ref[...].astype(jnp.bfloat16),
                fW_ref[2 * D:3 * D, :].astype(jnp.bfloat16),
                preferred_element_type=jnp.float32)
    acc_ref[...] += jnp.maximum(x + a_ref[...], 0.0)

    @pl.when(c == C - 1)
    def _():
        hs = acc_ref[...] * (1.0 / C)
        mu_ref[...] = (jnp.dot(hs, uW_ref[...],
                               preferred_element_type=jnp.float32)
                       + ub_ref[...])
        sig_ref[...] = _softplus(
            jnp.dot(hs, vW_ref[...], preferred_element_type=jnp.float32)
            + vb_ref[...]) + 1e-3


def _dec_body(e_ref, ct_ref, onehot_ref, memb_ref, uW_ref, ub_ref, vW_ref,
              vb_ref, sfmu_ref, sfsig_ref, nout_ref, score_ref):
    s_all = e_ref[0]
    for l in range(1, L):
        s_all = s_all + e_ref[l]
    m = jnp.dot(onehot_ref[...], memb_ref[...],
                preferred_element_type=jnp.float32)
    mmu = jnp.dot(m, uW_ref[D:2 * D, :], preferred_element_type=jnp.float32)
    mv = jnp.dot(m, vW_ref[D:2 * D, :], preferred_element_type=jnp.float32)
    sfmu = sfmu_ref[...]
    var_q = sfsig_ref[...] ** 2
    logvq = jnp.log(var_q)
    nout = nout_ref[...]
    cols = []
    for o in range(O):
        norm = jnp.maximum(ct_ref[:, o:o + 1], 1.0)
        s = s_all[o] / norm
        mu = (jnp.dot(s, uW_ref[0:D, :], preferred_element_type=jnp.float32)
              + mmu + ub_ref[...])
        sg = _softplus(
            jnp.dot(s, vW_ref[0:D, :], preferred_element_type=jnp.float32)
            + mv + vb_ref[...]) + 1e-3
        var_p = sg * sg
        dsq = jnp.sum((mu - sfmu) ** 2, axis=1, keepdims=True)
        kl = 0.5 * (float(D) * (jnp.log(var_p) - logvq)
                    + (float(D) * var_q + dsq) / var_p - float(D))
        cols.append(jnp.where(nout <= o, -jnp.inf, -kl))
    score_ref[...] = jnp.concatenate(cols, axis=1)


def kernel(sf_ids, metadata_ids, context_ids, lf_ids, target_lf_ids,
           lf_token_ct, global_ids, global_token_ct, lf_metadata_p,
           num_outputs, token_emb, enc_meta_emb, enc_fW, enc_fb, enc_uW,
           enc_ub, enc_vW, enc_vb, dec_token_emb, dec_meta_emb, dec_uW,
           dec_ub, dec_vW, dec_vb):
    ctx, e, center = _sc_gather(token_emb, dec_token_emb, context_ids,
                                lf_ids, sf_ids)
    onehot = jax.nn.one_hot(metadata_ids, NMETA, dtype=jnp.float32)

    mu, sig = pl.pallas_call(
        _enc_body,
        grid=(C,),
        in_specs=[
            pl.BlockSpec((None, B, D), lambda c: (c, 0, 0)),
            pl.BlockSpec((B, D), lambda c: (0, 0)),
            pl.BlockSpec((B, NMETA), lambda c: (0, 0)),
            pl.BlockSpec((3 * D, H), lambda c: (0, 0)),
            pl.BlockSpec((NMETA, D), lambda c: (0, 0)),
            pl.BlockSpec((1, H), lambda c: (0, 0)),
            pl.BlockSpec((H, D), lambda c: (0, 0)),
            pl.BlockSpec((1, D), lambda c: (0, 0)),
            pl.BlockSpec((H, 1), lambda c: (0, 0)),
            pl.BlockSpec((1, 1), lambda c: (0, 0)),
        ],
        out_specs=[
            pl.BlockSpec((B, D), lambda c: (0, 0)),
            pl.BlockSpec((B, 1), lambda c: (0, 0)),
        ],
        out_shape=[
            jax.ShapeDtypeStruct((B, D), jnp.float32),
            jax.ShapeDtypeStruct((B, 1), jnp.float32),
        ],
        scratch_shapes=[
            pltpu.VMEM((B, H), jnp.float32),
            pltpu.VMEM((B, H), jnp.float32),
        ],
    )(ctx, center, onehot, enc_fW, enc_meta_emb, enc_fb.reshape(1, H),
      enc_uW, enc_ub.reshape(1, D), enc_vW, enc_vb.reshape(1, 1))

    Bb = 256
    score = pl.pallas_call(
        _dec_body,
        grid=(B // Bb,),
        in_specs=[
            pl.BlockSpec((L, O, Bb, D), lambda i: (0, 0, i, 0)),
            pl.BlockSpec((Bb, O), lambda i: (i, 0)),
            pl.BlockSpec((Bb, NMETA), lambda i: (i, 0)),
            pl.BlockSpec((NMETA, D), lambda i: (0, 0)),
            pl.BlockSpec((2 * D, D), lambda i: (0, 0)),
            pl.BlockSpec((1, D), lambda i: (0, 0)),
            pl.BlockSpec((2 * D, 1), lambda i: (0, 0)),
            pl.BlockSpec((1, 1), lambda i: (0, 0)),
            pl.BlockSpec((Bb, D), lambda i: (i, 0)),
            pl.BlockSpec((Bb, 1), lambda i: (i, 0)),
            pl.BlockSpec((Bb, 1), lambda i: (i, 0)),
        ],
        out_specs=pl.BlockSpec((Bb, O), lambda i: (i, 0)),
        out_shape=jax.ShapeDtypeStruct((B, O), jnp.float32),
    )(e, lf_token_ct, onehot, dec_meta_emb, dec_uW, dec_ub.reshape(1, D),
      dec_vW, dec_vb.reshape(1, 1), mu, sig,
      num_outputs.reshape(B, 1).astype(jnp.int32))

    return (score, target_lf_ids)


# speedup vs baseline: 1.0797x; 1.0797x over previous
import functools

import jax
import jax.numpy as jnp
from jax import lax
from jax.experimental import pallas as pl
from jax.experimental.pallas import tpu as pltpu
from jax.experimental.pallas import tpu_sc as plsc

B, C, O, L = 1024, 50, 10, 5
NMETA, D, H = 10, 128, 256

NC, NS = 2, 16
NW = NC * NS
CHUNK = 80
NCH = (C * B) // (NW * CHUNK)
CEN_PER_W = B // NW


def _pipe_gather(idx, table, out, base, nch, rows, buf, gsem, wsem):
    gets, puts = [], []
    gets.append(pltpu.async_copy(
        table.at[idx.at[0]], buf.at[0, pl.ds(0, rows)], gsem))
    for k in range(nch):
        gets[k].wait()
        puts.append(pltpu.async_copy(
            buf.at[k % 2, pl.ds(0, rows)],
            out.at[pl.ds(base + k * rows, rows)], wsem))
        if k + 1 < nch:
            if k >= 1:
                puts[k - 1].wait()
            gets.append(pltpu.async_copy(
                table.at[idx.at[k + 1]],
                buf.at[(k + 1) % 2, pl.ds(0, rows)], gsem))
    if nch >= 2:
        puts[nch - 2].wait()
    puts[nch - 1].wait()


def _ctx_body(temb_ref, cidx_ref, sidx_ref, ctx_out, cen_out,
              idx_v, sidx_v, buf, gsem, wsem):
    wid = lax.axis_index("s") * NC + lax.axis_index("c")
    pltpu.sync_copy(cidx_ref.at[wid], idx_v)
    pltpu.sync_copy(sidx_ref.at[wid], sidx_v)
    _pipe_gather(idx_v, temb_ref, ctx_out, wid * CHUNK * NCH, NCH, CHUNK,
                 buf, gsem, wsem)
    _pipe_gather(sidx_v, temb_ref, cen_out, wid * CEN_PER_W, 1, CEN_PER_W,
                 buf, gsem, wsem)


def _dec_gather_body(demb_ref, didx_ref, dec_out, idx_v, buf, gsem, wsem):
    wid = lax.axis_index("s") * NC + lax.axis_index("c")
    pltpu.sync_copy(didx_ref.at[wid], idx_v)
    _pipe_gather(idx_v, demb_ref, dec_out, wid * CHUNK * NCH, NCH, CHUNK,
                 buf, gsem, wsem)


def _sc_mesh():
    return plsc.VectorSubcoreMesh(core_axis_name="c", subcore_axis_name="s",
                                  num_cores=NC, num_subcores=NS)


def _sc_gather(token_emb, dec_token_emb, context_ids, lf_ids, sf_ids):
    cidx = jnp.transpose(context_ids).reshape(NW, NCH, CHUNK).astype(jnp.int32)
    didx = jnp.transpose(lf_ids, (2, 1, 0)).reshape(NW, NCH, CHUNK).astype(jnp.int32)
    sidx = sf_ids.reshape(NW, 1, CEN_PER_W).astype(jnp.int32)

    ctx_out, cen_out = pl.kernel(
        _ctx_body,
        out_type=[
            jax.ShapeDtypeStruct((C * B, D), jnp.float32),
            jax.ShapeDtypeStruct((B, D), jnp.float32),
        ],
        mesh=_sc_mesh(),
        scratch_types=[
            pltpu.VMEM((NCH, CHUNK), jnp.int32),
            pltpu.VMEM((1, CEN_PER_W), jnp.int32),
            pltpu.VMEM((2, CHUNK, D), jnp.float32),
            pltpu.SemaphoreType.DMA,
            pltpu.SemaphoreType.DMA,
        ],
    )(token_emb, cidx, sidx)

    dec_out = pl.kernel(
        _dec_gather_body,
        out_type=jax.ShapeDtypeStruct((L * O * B, D), jnp.float32),
        mesh=_sc_mesh(),
        scratch_types=[
            pltpu.VMEM((NCH, CHUNK), jnp.int32),
            pltpu.VMEM((2, CHUNK, D), jnp.float32),
            pltpu.SemaphoreType.DMA,
            pltpu.SemaphoreType.DMA,
        ],
    )(dec_token_emb, didx)
    return (ctx_out.reshape(C, B, D), dec_out.reshape(L, O, B, D), cen_out)


def _softplus(x):
    return jnp.maximum(x, 0.0) + jnp.log(1.0 + jnp.exp(-jnp.abs(x)))


NCC = 5


def _enc_body(ctx_ref, center_ref, onehot_ref, fW_ref, memb_ref, fb_ref,
              uW_ref, ub_ref, vW_ref, vb_ref, mu_ref, sig_ref, a_ref, acc_ref):
    c = pl.program_id(0)

    @pl.when(c == 0)
    def _():
        m = jnp.dot(onehot_ref[...], memb_ref[...],
                    preferred_element_type=jnp.float32)
        a_ref[...] = (jnp.dot(center_ref[...], fW_ref[0:D, :],
                              preferred_element_type=jnp.float32)
                      + jnp.dot(m, fW_ref[D:2 * D, :],
                                preferred_element_type=jnp.float32)
                      + fb_ref[...])
        acc_ref[...] = jnp.zeros_like(acc_ref)

    Wx = fW_ref[2 * D:3 * D, :].astype(jnp.bfloat16)
    a = a_ref[...]
    t = jnp.maximum(
        jnp.dot(ctx_ref[0].astype(jnp.bfloat16), Wx,
                preferred_element_type=jnp.float32) + a, 0.0)
    for j in range(1, NCC):
        t += jnp.maximum(
            jnp.dot(ctx_ref[j].astype(jnp.bfloat16), Wx,
                    preferred_element_type=jnp.float32) + a, 0.0)
    acc_ref[...] += t

    @pl.when(c == C // NCC - 1)
    def _():
        hs = acc_ref[...] * (1.0 / C)
        mu_ref[...] = (jnp.dot(hs, uW_ref[...],
                               preferred_element_type=jnp.float32)
                       + ub_ref[...])
        sig_ref[...] = _softplus(
            jnp.dot(hs, vW_ref[...], preferred_element_type=jnp.float32)
            + vb_ref[...]) + 1e-3


def _dec_body(e_ref, ct_ref, onehot_ref, memb_ref, uW_ref, ub_ref, vW_ref,
              vb_ref, sfmu_ref, sfsig_ref, nout_ref, score_ref):
    s_all = e_ref[0]
    for l in range(1, L):
        s_all = s_all + e_ref[l]
    m = jnp.dot(onehot_ref[...], memb_ref[...],
                preferred_element_type=jnp.float32)
    mmu = jnp.dot(m, uW_ref[D:2 * D, :], preferred_element_type=jnp.float32)
    mv = jnp.dot(m, vW_ref[D:2 * D, :], preferred_element_type=jnp.float32)
    sfmu = sfmu_ref[...]
    var_q = sfsig_ref[...] ** 2
    logvq = jnp.log(var_q)
    nout = nout_ref[...]
    cols = []
    for o in range(O):
        norm = jnp.maximum(ct_ref[:, o:o + 1], 1.0)
        s = s_all[o] / norm
        mu = (jnp.dot(s, uW_ref[0:D, :], preferred_element_type=jnp.float32)
              + mmu + ub_ref[...])
        sg = _softplus(
            jnp.dot(s, vW_ref[0:D, :], preferred_element_type=jnp.float32)
            + mv + vb_ref[...]) + 1e-3
        var_p = sg * sg
        dsq = jnp.sum((mu - sfmu) ** 2, axis=1, keepdims=True)
        kl = 0.5 * (float(D) * (jnp.log(var_p) - logvq)
                    + (float(D) * var_q + dsq) / var_p - float(D))
        cols.append(jnp.where(nout <= o, -jnp.inf, -kl))
    score_ref[...] = jnp.concatenate(cols, axis=1)


def kernel(sf_ids, metadata_ids, context_ids, lf_ids, target_lf_ids,
           lf_token_ct, global_ids, global_token_ct, lf_metadata_p,
           num_outputs, token_emb, enc_meta_emb, enc_fW, enc_fb, enc_uW,
           enc_ub, enc_vW, enc_vb, dec_token_emb, dec_meta_emb, dec_uW,
           dec_ub, dec_vW, dec_vb):
    ctx, e, center = _sc_gather(token_emb, dec_token_emb, context_ids,
                                lf_ids, sf_ids)
    onehot = jax.nn.one_hot(metadata_ids, NMETA, dtype=jnp.float32)

    mu, sig = pl.pallas_call(
        _enc_body,
        grid=(C // NCC,),
        in_specs=[
            pl.BlockSpec((NCC, B, D), lambda c: (c, 0, 0)),
            pl.BlockSpec((B, D), lambda c: (0, 0)),
            pl.BlockSpec((B, NMETA), lambda c: (0, 0)),
            pl.BlockSpec((3 * D, H), lambda c: (0, 0)),
            pl.BlockSpec((NMETA, D), lambda c: (0, 0)),
            pl.BlockSpec((1, H), lambda c: (0, 0)),
            pl.BlockSpec((H, D), lambda c: (0, 0)),
            pl.BlockSpec((1, D), lambda c: (0, 0)),
            pl.BlockSpec((H, 1), lambda c: (0, 0)),
            pl.BlockSpec((1, 1), lambda c: (0, 0)),
        ],
        out_specs=[
            pl.BlockSpec((B, D), lambda c: (0, 0)),
            pl.BlockSpec((B, 1), lambda c: (0, 0)),
        ],
        out_shape=[
            jax.ShapeDtypeStruct((B, D), jnp.float32),
            jax.ShapeDtypeStruct((B, 1), jnp.float32),
        ],
        scratch_shapes=[
            pltpu.VMEM((B, H), jnp.float32),
            pltpu.VMEM((B, H), jnp.float32),
        ],
    )(ctx, center, onehot, enc_fW, enc_meta_emb, enc_fb.reshape(1, H),
      enc_uW, enc_ub.reshape(1, D), enc_vW, enc_vb.reshape(1, 1))

    Bb = 256
    score = pl.pallas_call(
        _dec_body,
        grid=(B // Bb,),
        in_specs=[
            pl.BlockSpec((L, O, Bb, D), lambda i: (0, 0, i, 0)),
            pl.BlockSpec((Bb, O), lambda i: (i, 0)),
            pl.BlockSpec((Bb, NMETA), lambda i: (i, 0)),
            pl.BlockSpec((NMETA, D), lambda i: (0, 0)),
            pl.BlockSpec((2 * D, D), lambda i: (0, 0)),
            pl.BlockSpec((1, D), lambda i: (0, 0)),
            pl.BlockSpec((2 * D, 1), lambda i: (0, 0)),
            pl.BlockSpec((1, 1), lambda i: (0, 0)),
            pl.BlockSpec((Bb, D), lambda i: (i, 0)),
            pl.BlockSpec((Bb, 1), lambda i: (i, 0)),
            pl.BlockSpec((Bb, 1), lambda i: (i, 0)),
        ],
        out_specs=pl.BlockSpec((Bb, O), lambda i: (i, 0)),
        out_shape=jax.ShapeDtypeStruct((B, O), jnp.float32),
    )(e, lf_token_ct, onehot, dec_meta_emb, dec_uW, dec_ub.reshape(1, D),
      dec_vW, dec_vb.reshape(1, 1), mu, sig,
      num_outputs.reshape(B, 1).astype(jnp.int32))

    return (score, target_lf_ids)
